# baseline (speedup 1.0000x reference)
import jax
import jax.numpy as jnp
from jax.experimental import pallas as pl
from jax.experimental.pallas import tpu as pltpu

_TN = 128


def _fused_body(x_ref, b1_ref, bias1_ref, b2_ref, bias2_ref, s_ref,
                wfc1_ref, bfc1_ref, wfc2_ref, bfc2_ref, o_ref):
    tn = _TN
    rq = tn * 7
    r1 = rq - 1
    r2 = rq - 3

    xb = x_ref[...]
    xw = jnp.concatenate([xb[0:r1], xb[1:r1 + 1]], axis=1)
    y1 = jnp.dot(xw, b1_ref[...], preferred_element_type=jnp.float32)

    p1 = jnp.maximum(y1[:, 0:512], y1[:, 512:1024])
    p2 = jnp.maximum(y1[:, 1024:1536], y1[:, 1536:2048])
    x1e = jnp.maximum(jnp.maximum(p1[:, 0:240], p1[:, 256:496])
                      + bias1_ref[...], 0.0).astype(jnp.bfloat16)
    x1o = jnp.maximum(jnp.maximum(p2[:, 0:240], p2[:, 256:496])
                      + bias1_ref[...], 0.0).astype(jnp.bfloat16)

    xw3 = jnp.concatenate(
        [x1e[0:r2], x1o[0:r2], x1e[1:r2 + 1], x1o[1:r2 + 1],
         x1e[2:r2 + 2], x1o[2:r2 + 2]], axis=1)
    y2 = jnp.dot(xw3, b2_ref[...], preferred_element_type=jnp.float32)

    v2 = jnp.maximum(y2[:, 0:512], y2[:, 512:1024])
    x2 = jnp.maximum(jnp.maximum(v2[:, 0:200], v2[:, 256:456])
                     + bias2_ref[...], 0.0).astype(jnp.bfloat16)

    a = jnp.dot(s_ref[...], x2,
                preferred_element_type=jnp.float32).astype(jnp.bfloat16)
    h = None
    for i3 in range(4):
        t = jnp.dot(a[i3 * tn:(i3 + 1) * tn], wfc1_ref[i3],
                    preferred_element_type=jnp.float32)
        h = t if h is None else h + t
    h = jnp.maximum(h + bfc1_ref[...], 0.0).astype(jnp.bfloat16)
    logits = jnp.dot(h, wfc2_ref[...], preferred_element_type=jnp.float32)
    o_ref[...] = logits + bfc2_ref[...]


def _prep_weights(b1w, b2w, tn):
    b1h = b1w.astype(jnp.bfloat16)
    b2h = b2w.astype(jnp.bfloat16)
    z1 = jnp.zeros((28, 512), jnp.bfloat16)
    z2 = jnp.zeros((240, 512), jnp.bfloat16)
    b1 = jnp.concatenate(
        [jnp.concatenate([b1h[o - r] if 0 <= o - r <= 4 else z1
                          for r in range(4)], axis=1)
         for o in range(8)], axis=0)
    b2 = jnp.concatenate(
        [jnp.concatenate([b2h[o - r] if 0 <= o - r <= 4 else z2
                          for r in range(2)], axis=1)
         for o in range(6)], axis=0)
    g = jnp.arange(4 * tn)
    col = 7 * (g % tn) + g // tn
    s = (jnp.arange(tn * 7 - 3)[None, :] == col[:, None]).astype(jnp.bfloat16)
    return b1, b2, s


def kernel(b1w, bias1, b2w, bias2, wfc1, bfc1, wfc2, bfc2, x):
    n = x.shape[0]
    tn = _TN
    npad = ((n + tn - 1) // tn) * tn
    xr = x.astype(jnp.float32).reshape(n, 28, 28)
    if npad != n:
        xr = jnp.concatenate(
            [xr, jnp.zeros((npad - n, 28, 28), jnp.float32)], axis=0)
    xq = xr.reshape(npad * 7, 112).astype(jnp.bfloat16)

    b1, b2, s = _prep_weights(b1w, b2w, tn)
    w1r = wfc1.astype(jnp.bfloat16)
    w2r = wfc2.astype(jnp.bfloat16)

    steps = npad // tn
    rq = tn * 7
    out = pl.pallas_call(
        _fused_body,
        out_shape=jax.ShapeDtypeStruct((npad, 10), jnp.float32),
        grid=(steps,),
        in_specs=[
            pl.BlockSpec((rq, 112), lambda i: (i, 0)),
            pl.BlockSpec((224, 2048), lambda i: (0, 0)),
            pl.BlockSpec((1, 240), lambda i: (0, 0)),
            pl.BlockSpec((1440, 1024), lambda i: (0, 0)),
            pl.BlockSpec((1, 200), lambda i: (0, 0)),
            pl.BlockSpec((4 * tn, rq - 3), lambda i: (0, 0)),
            pl.BlockSpec((4, 200, 500), lambda i: (0, 0, 0)),
            pl.BlockSpec((1, 500), lambda i: (0, 0)),
            pl.BlockSpec((500, 10), lambda i: (0, 0)),
            pl.BlockSpec((1, 10), lambda i: (0, 0)),
        ],
        out_specs=pl.BlockSpec((tn, 10), lambda i: (i, 0)),
        compiler_params=pltpu.CompilerParams(
            dimension_semantics=("parallel",),
            vmem_limit_bytes=64 * 1024 * 1024),
    )(xq, b1, bias1, b2, bias2, s, w1r, bfc1, w2r, bfc2)

    return out[:n] if npad != n else out


# speedup vs baseline: 2.6548x; 2.6548x over previous
import jax
import jax.numpy as jnp
from jax.experimental import pallas as pl
from jax.experimental.pallas import tpu as pltpu

_TN = 128


def _fused_body(x_ref, b1_ref, bias1_ref, b2_ref, bias2_ref, s_ref,
                wfc1_ref, bfc1_ref, wfc2_ref, bfc2_ref, o_ref):
    tn = _TN
    rq = tn * 7
    r1 = rq - 1
    r2 = rq - 3

    xb = x_ref[...].astype(jnp.bfloat16)
    xw = jnp.concatenate([xb[0:r1], xb[1:r1 + 1]], axis=1)
    y1 = jnp.dot(xw, b1_ref[...], preferred_element_type=jnp.float32)

    p1 = jnp.maximum(y1[:, 0:512], y1[:, 512:1024])
    p2 = jnp.maximum(y1[:, 1024:1536], y1[:, 1536:2048])
    x1e = jnp.maximum(jnp.maximum(p1[:, 0:240], p1[:, 256:496])
                      + bias1_ref[...], 0.0).astype(jnp.bfloat16)
    x1o = jnp.maximum(jnp.maximum(p2[:, 0:240], p2[:, 256:496])
                      + bias1_ref[...], 0.0).astype(jnp.bfloat16)

    xw3 = jnp.concatenate(
        [x1e[0:r2], x1o[0:r2], x1e[1:r2 + 1], x1o[1:r2 + 1],
         x1e[2:r2 + 2], x1o[2:r2 + 2]], axis=1)
    y2 = jnp.dot(xw3, b2_ref[...], preferred_element_type=jnp.float32)

    v2 = jnp.maximum(y2[:, 0:512], y2[:, 512:1024])
    x2 = jnp.maximum(jnp.maximum(v2[:, 0:200], v2[:, 256:456])
                     + bias2_ref[...], 0.0).astype(jnp.bfloat16)

    a = jnp.dot(s_ref[...], x2,
                preferred_element_type=jnp.float32).astype(jnp.bfloat16)
    h = None
    for i3 in range(4):
        t = jnp.dot(a[i3 * tn:(i3 + 1) * tn], wfc1_ref[i3],
                    preferred_element_type=jnp.float32)
        h = t if h is None else h + t
    h = jnp.maximum(h + bfc1_ref[...], 0.0).astype(jnp.bfloat16)
    logits = jnp.dot(h, wfc2_ref[...], preferred_element_type=jnp.float32)
    o_ref[...] = logits + bfc2_ref[...]


def _prep_weights(b1w, b2w, tn):
    b1h = b1w.astype(jnp.bfloat16)
    b2h = b2w.astype(jnp.bfloat16)
    z1 = jnp.zeros((28, 512), jnp.bfloat16)
    z2 = jnp.zeros((240, 512), jnp.bfloat16)
    b1 = jnp.concatenate(
        [jnp.concatenate([b1h[o - r] if 0 <= o - r <= 4 else z1
                          for r in range(4)], axis=1)
         for o in range(8)], axis=0)
    b2 = jnp.concatenate(
        [jnp.concatenate([b2h[o - r] if 0 <= o - r <= 4 else z2
                          for r in range(2)], axis=1)
         for o in range(6)], axis=0)
    g = jnp.arange(4 * tn)
    col = 7 * (g % tn) + g // tn
    s = (jnp.arange(tn * 7 - 3)[None, :] == col[:, None]).astype(jnp.bfloat16)
    return b1, b2, s


def kernel(b1w, bias1, b2w, bias2, wfc1, bfc1, wfc2, bfc2, x):
    n = x.shape[0]
    tn = _TN
    npad = ((n + tn - 1) // tn) * tn
    xr = x.astype(jnp.float32).reshape(n, 28, 28)
    if npad != n:
        xr = jnp.concatenate(
            [xr, jnp.zeros((npad - n, 28, 28), jnp.float32)], axis=0)
    xq = xr.reshape(npad * 7, 112)

    b1, b2, s = _prep_weights(b1w, b2w, tn)
    w1r = wfc1.astype(jnp.bfloat16)
    w2r = wfc2.astype(jnp.bfloat16)

    steps = npad // tn
    rq = tn * 7
    out = pl.pallas_call(
        _fused_body,
        out_shape=jax.ShapeDtypeStruct((npad, 10), jnp.float32),
        grid=(steps,),
        in_specs=[
            pl.BlockSpec((rq, 112), lambda i: (i, 0)),
            pl.BlockSpec((224, 2048), lambda i: (0, 0)),
            pl.BlockSpec((1, 240), lambda i: (0, 0)),
            pl.BlockSpec((1440, 1024), lambda i: (0, 0)),
            pl.BlockSpec((1, 200), lambda i: (0, 0)),
            pl.BlockSpec((4 * tn, rq - 3), lambda i: (0, 0)),
            pl.BlockSpec((4, 200, 500), lambda i: (0, 0, 0)),
            pl.BlockSpec((1, 500), lambda i: (0, 0)),
            pl.BlockSpec((500, 10), lambda i: (0, 0)),
            pl.BlockSpec((1, 10), lambda i: (0, 0)),
        ],
        out_specs=pl.BlockSpec((tn, 10), lambda i: (i, 0)),
        compiler_params=pltpu.CompilerParams(
            dimension_semantics=("parallel",),
            vmem_limit_bytes=64 * 1024 * 1024),
    )(xq, b1, bias1, b2, bias2, s, w1r, bfc1, w2r, bfc2)

    return out[:n] if npad != n else out


# speedup vs baseline: 2.6716x; 1.0063x over previous
import jax
import jax.numpy as jnp
from jax.experimental import pallas as pl
from jax.experimental.pallas import tpu as pltpu

_TN = 128


def _fused_body(x_ref, b1_ref, bias1_ref, b2_ref, bias2_ref, s_ref,
                wfc1_ref, bfc1_ref, wfc2_ref, bfc2_ref, o_ref):
    tn = _TN
    rq = tn * 7
    r1 = rq - 1
    r2 = rq - 3
    bf = jnp.bfloat16

    xb = jnp.pad(x_ref[...].astype(bf), ((0, 0), (0, 16)))
    xw = jnp.concatenate([xb[0:r1], xb[1:r1 + 1]], axis=1)
    y1 = jnp.dot(xw, b1_ref[...], preferred_element_type=jnp.float32)

    p1 = jnp.maximum(y1[:, 0:512], y1[:, 512:1024])
    p2 = jnp.maximum(y1[:, 1024:1536], y1[:, 1536:2048])
    x1e = jnp.maximum(jnp.maximum(p1[:, 0:256], p1[:, 256:512])
                      + bias1_ref[...], 0.0).astype(bf)
    x1o = jnp.maximum(jnp.maximum(p2[:, 0:256], p2[:, 256:512])
                      + bias1_ref[...], 0.0).astype(bf)

    xw3 = jnp.concatenate(
        [x1e[0:r2], x1o[0:r2], x1e[1:r2 + 1], x1o[1:r2 + 1],
         x1e[2:r2 + 2], x1o[2:r2 + 2]], axis=1)
    y2 = jnp.dot(xw3, b2_ref[...], preferred_element_type=jnp.float32)

    v2 = jnp.maximum(y2[:, 0:512], y2[:, 512:1024])
    x2 = jnp.maximum(jnp.maximum(v2[:, 0:256], v2[:, 256:512])
                     + bias2_ref[...], 0.0).astype(bf)

    a = jnp.dot(s_ref[...], x2,
                preferred_element_type=jnp.float32).astype(bf)
    af = jnp.concatenate([a[i3 * tn:(i3 + 1) * tn] for i3 in range(4)],
                         axis=1)
    h = jnp.dot(af, wfc1_ref[...], preferred_element_type=jnp.float32)
    h = jnp.maximum(h + bfc1_ref[...], 0.0).astype(bf)
    logits = jnp.dot(h, wfc2_ref[...], preferred_element_type=jnp.float32)
    o_ref[...] = logits + bfc2_ref[...]


def _prep_weights(b1w, b2w, wfc1, tn):
    bf = jnp.bfloat16
    b1h = b1w.astype(bf)
    b2h = jnp.pad(b2w.astype(bf), ((0, 0), (0, 16), (0, 0)))
    z1 = jnp.zeros((28, 512), bf)
    z2 = jnp.zeros((256, 512), bf)
    z16 = jnp.zeros((16, 2048), bf)
    halves = []
    for u in range(2):
        blocks = [jnp.concatenate([b1h[o - r] if 0 <= o - r <= 4 else z1
                                   for r in range(4)], axis=1)
                  for o in range(4 * u, 4 * u + 4)]
        halves.append(jnp.concatenate(blocks + [z16], axis=0))
    b1 = jnp.concatenate(halves, axis=0)
    b2 = jnp.concatenate(
        [jnp.concatenate([b2h[o - r] if 0 <= o - r <= 4 else z2
                          for r in range(2)], axis=1)
         for o in range(6)], axis=0)
    g = jnp.arange(4 * tn)
    col = 7 * (g % tn) + g // tn
    s = (jnp.arange(tn * 7 - 3)[None, :] == col[:, None]).astype(bf)
    w1r = jnp.pad(wfc1.astype(bf),
                  ((0, 0), (0, 56), (0, 0))).reshape(1024, 500)
    return b1, b2, s, w1r


def kernel(b1w, bias1, b2w, bias2, wfc1, bfc1, wfc2, bfc2, x):
    n = x.shape[0]
    tn = _TN
    npad = ((n + tn - 1) // tn) * tn
    xr = x.astype(jnp.float32).reshape(n, 28, 28)
    if npad != n:
        xr = jnp.concatenate(
            [xr, jnp.zeros((npad - n, 28, 28), jnp.float32)], axis=0)
    xq = xr.reshape(npad * 7, 112)

    b1, b2, s, w1r = _prep_weights(b1w, b2w, wfc1, tn)
    bias1p = jnp.pad(bias1, ((0, 0), (0, 16)))
    bias2p = jnp.pad(bias2, ((0, 0), (0, 56)))
    w2r = wfc2.astype(jnp.bfloat16)

    steps = npad // tn
    rq = tn * 7
    out = pl.pallas_call(
        _fused_body,
        out_shape=jax.ShapeDtypeStruct((npad, 10), jnp.float32),
        grid=(steps,),
        in_specs=[
            pl.BlockSpec((rq, 112), lambda i: (i, 0)),
            pl.BlockSpec((256, 2048), lambda i: (0, 0)),
            pl.BlockSpec((1, 256), lambda i: (0, 0)),
            pl.BlockSpec((1536, 1024), lambda i: (0, 0)),
            pl.BlockSpec((1, 256), lambda i: (0, 0)),
            pl.BlockSpec((4 * tn, rq - 3), lambda i: (0, 0)),
            pl.BlockSpec((1024, 500), lambda i: (0, 0)),
            pl.BlockSpec((1, 500), lambda i: (0, 0)),
            pl.BlockSpec((500, 10), lambda i: (0, 0)),
            pl.BlockSpec((1, 10), lambda i: (0, 0)),
        ],
        out_specs=pl.BlockSpec((tn, 10), lambda i: (i, 0)),
        compiler_params=pltpu.CompilerParams(
            dimension_semantics=("parallel",),
            vmem_limit_bytes=64 * 1024 * 1024),
    )(xq, b1, bias1p, b2, bias2p, s, w1r, bfc1, w2r, bfc2)

    return out[:n] if npad != n else out
